# baseline (speedup 1.0000x reference)
import functools

import jax
import jax.numpy as jnp
from jax import lax
from jax.experimental import pallas as pl
from jax.experimental.pallas import tpu as pltpu
from jax.experimental.pallas import tpu_sc as plsc

NC = 2
NS = 16
NW = NC * NS
CH = 128
NBUF = 4


@functools.lru_cache(maxsize=None)
def _build_sc_kernel(B, T, D):
    b_per_w = B // NW
    mchunks = b_per_w // CH
    t_per_w = (T - B) // NS
    tchunks = t_per_w // CH
    ngroups = tchunks // NBUF

    mesh = plsc.VectorSubcoreMesh(core_axis_name="c", subcore_axis_name="s")

    @functools.partial(
        pl.kernel,
        out_type=jax.ShapeDtypeStruct((B, D), jnp.float32),
        mesh=mesh,
        scratch_types=[
            pltpu.VMEM((b_per_w,), jnp.int32),
            pltpu.VMEM((b_per_w,), jnp.int32),
            pltpu.VMEM((t_per_w,), jnp.int32),
            pltpu.VMEM((t_per_w,), jnp.int32),
            pltpu.VMEM((b_per_w, D), jnp.float32),
            pltpu.VMEM((CH, D), jnp.float32),
            pltpu.VMEM((CH, D), jnp.float32),
            pltpu.VMEM((CH, D), jnp.float32),
            pltpu.VMEM((CH, D), jnp.float32),
            pltpu.VMEM((D,), jnp.float32),
            pltpu.VMEM((NS, 1, D), jnp.float32),
            pltpu.VMEM((1, D), jnp.float32),
            pltpu.VMEM_SHARED((NS, 1, D), jnp.float32),
            pltpu.SemaphoreType.DMA,
            pltpu.SemaphoreType.DMA,
            pltpu.SemaphoreType.DMA,
            pltpu.SemaphoreType.DMA,
        ],
        compiler_params=pltpu.CompilerParams(use_tc_tiling_on_sc=False),
    )
    def sc_kernel(text_hbm, deps_hbm, wt_hbm, wd_hbm, bias_hbm,
                  out_hbm,
                  idx_mt, idx_md, idx_tt, idx_td, outb,
                  acc0, acc1, acc2, acc3, bias_v, partv, stage, spm,
                  sem0, sem1, sem2, sem3):
        accs = (acc0, acc1, acc2, acc3)
        sems = (sem0, sem1, sem2, sem3)
        cid = lax.axis_index("c")
        sid = lax.axis_index("s")
        w = sid * NC + cid
        mb = NW - 1 - w
        is_fixer = w == 0

        pltpu.sync_copy(bias_hbm, bias_v)
        pltpu.sync_copy(text_hbm.at[pl.ds(mb * b_per_w, b_per_w)], idx_mt)
        pltpu.sync_copy(deps_hbm.at[pl.ds(mb * b_per_w, b_per_w)], idx_md)

        bv = bias_v[...]

        def init_main(i, carry):
            outb[i] = bv
            return carry

        lax.fori_loop(0, b_per_w, init_main, 0)

        for j in range(mchunks):
            pltpu.async_copy(wt_hbm.at[idx_mt.at[pl.ds(j * CH, CH)]],
                             outb.at[pl.ds(j * CH, CH)], sems[j % NBUF],
                             add=True)
        for j in range(mchunks):
            pltpu.make_async_copy(wt_hbm.at[idx_mt.at[pl.ds(j * CH, CH)]],
                                  outb.at[pl.ds(j * CH, CH)],
                                  sems[j % NBUF]).wait()
        for j in range(mchunks):
            pltpu.async_copy(wd_hbm.at[idx_md.at[pl.ds(j * CH, CH)]],
                             outb.at[pl.ds(j * CH, CH)], sems[j % NBUF],
                             add=True)
        for j in range(mchunks):
            pltpu.make_async_copy(wd_hbm.at[idx_md.at[pl.ds(j * CH, CH)]],
                                  outb.at[pl.ds(j * CH, CH)],
                                  sems[j % NBUF]).wait()

        @pl.when(jnp.logical_not(is_fixer))
        def _():
            pltpu.sync_copy(outb, out_hbm.at[pl.ds(mb * b_per_w, b_per_w)])

        @pl.when(cid == 0)
        def _():
            pltpu.sync_copy(text_hbm.at[pl.ds(B + sid * t_per_w, t_per_w)],
                            idx_tt)
            pltpu.sync_copy(deps_hbm.at[pl.ds(B + sid * t_per_w, t_per_w)],
                            idx_td)

            zero = jnp.zeros((D,), jnp.float32)

            def init_acc(i, carry):
                for a in accs:
                    a[i] = zero
                return carry

            lax.fori_loop(0, CH, init_acc, 0)

            def run_table(src_hbm, idx_ref):
                def chunk_slice(c):
                    return idx_ref.at[pl.ds(pl.multiple_of(c * CH, CH), CH)]

                for b in range(NBUF):
                    pltpu.async_copy(src_hbm.at[chunk_slice(b)], accs[b],
                                     sems[b], add=True)

                def body(g, carry):
                    for b in range(NBUF):
                        pltpu.make_async_copy(src_hbm.at[chunk_slice(b)],
                                              accs[b], sems[b]).wait()
                        pltpu.async_copy(src_hbm.at[chunk_slice(g * NBUF + b)],
                                         accs[b], sems[b], add=True)
                    return carry

                lax.fori_loop(1, ngroups, body, 0)
                for b in range(NBUF):
                    pltpu.make_async_copy(src_hbm.at[chunk_slice(b)],
                                          accs[b], sems[b]).wait()

            run_table(wt_hbm, idx_tt)
            run_table(wd_hbm, idx_td)

            def red(i, carry):
                return carry + ((acc0[i] + acc1[i]) + (acc2[i] + acc3[i]))

            total = lax.fori_loop(0, CH, red, jnp.zeros((D,), jnp.float32))
            stage[0] = total
            pltpu.sync_copy(stage, spm.at[sid])

        plsc.subcore_barrier()

        @pl.when(is_fixer)
        def _():
            pltpu.sync_copy(spm, partv)

            def redp(i, carry):
                return carry + partv[i, 0, :]

            tail_total = lax.fori_loop(0, NS, redp,
                                       jnp.zeros((D,), jnp.float32))
            outb[b_per_w - 1] = outb[b_per_w - 1] + tail_total
            pltpu.sync_copy(outb, out_hbm.at[pl.ds(mb * b_per_w, b_per_w)])

    return sc_kernel


def kernel(text, text_offsets, deps, deps_offsets, W_text, W_deps, bias):
    B = text_offsets.shape[0]
    T = text.shape[0]
    D = W_text.shape[1]
    sc_kernel = _build_sc_kernel(B, T, D)
    return sc_kernel(text.astype(jnp.int32), deps.astype(jnp.int32),
                     W_text.astype(jnp.float32), W_deps.astype(jnp.float32),
                     bias.astype(jnp.float32))


# speedup vs baseline: 1.0004x; 1.0004x over previous
import functools

import jax
import jax.numpy as jnp
from jax import lax
from jax.experimental import pallas as pl
from jax.experimental.pallas import tpu as pltpu
from jax.experimental.pallas import tpu_sc as plsc

NC = 2
NS = 16
NW = NC * NS
CH = 128
NBUF = 4


@functools.lru_cache(maxsize=None)
def _build_sc_kernel(B, T, D):
    b_per_w = B // NW
    mchunks = b_per_w // CH
    t_per_w = (T - B) // NS
    tchunks = t_per_w // CH
    ngroups = tchunks // NBUF

    mesh = plsc.VectorSubcoreMesh(core_axis_name="c", subcore_axis_name="s")

    @functools.partial(
        pl.kernel,
        out_type=jax.ShapeDtypeStruct((B, D), jnp.float32),
        mesh=mesh,
        scratch_types=[
            pltpu.VMEM((b_per_w,), jnp.int32),
            pltpu.VMEM((b_per_w,), jnp.int32),
            pltpu.VMEM((t_per_w,), jnp.int32),
            pltpu.VMEM((t_per_w,), jnp.int32),
            pltpu.VMEM((b_per_w, D), jnp.float32),
            pltpu.VMEM((CH, D), jnp.float32),
            pltpu.VMEM((CH, D), jnp.float32),
            pltpu.VMEM((CH, D), jnp.float32),
            pltpu.VMEM((CH, D), jnp.float32),
            pltpu.VMEM((D,), jnp.float32),
            pltpu.VMEM((NS, 1, D), jnp.float32),
            pltpu.VMEM((1, D), jnp.float32),
            pltpu.VMEM_SHARED((NS, 1, D), jnp.float32),
            pltpu.SemaphoreType.DMA,
            pltpu.SemaphoreType.DMA,
            pltpu.SemaphoreType.DMA,
            pltpu.SemaphoreType.DMA,
        ],
        compiler_params=pltpu.CompilerParams(use_tc_tiling_on_sc=False),
    )
    def sc_kernel(text_hbm, deps_hbm, wt_hbm, wd_hbm, bias_hbm,
                  out_hbm,
                  idx_mt, idx_md, idx_tt, idx_td, outb,
                  acc0, acc1, acc2, acc3, bias_v, partv, stage, spm,
                  sem0, sem1, sem2, sem3):
        accs = (acc0, acc1, acc2, acc3)
        sems = (sem0, sem1, sem2, sem3)
        cid = lax.axis_index("c")
        sid = lax.axis_index("s")
        w = sid * NC + cid
        mb = NW - 1 - w
        is_fixer = w == 0

        pltpu.sync_copy(bias_hbm, bias_v)
        pltpu.sync_copy(text_hbm.at[pl.ds(mb * b_per_w, b_per_w)], idx_mt)
        pltpu.sync_copy(deps_hbm.at[pl.ds(mb * b_per_w, b_per_w)], idx_md)

        bv = bias_v[...]

        def init_main(i, carry):
            outb[i] = bv
            return carry

        lax.fori_loop(0, b_per_w, init_main, 0)

        for j in range(mchunks):
            pltpu.async_copy(wt_hbm.at[idx_mt.at[pl.ds(j * CH, CH)]],
                             outb.at[pl.ds(j * CH, CH)], sems[j % NBUF],
                             add=True)
        for j in range(mchunks):
            pltpu.make_async_copy(wt_hbm.at[idx_mt.at[pl.ds(j * CH, CH)]],
                                  outb.at[pl.ds(j * CH, CH)],
                                  sems[j % NBUF]).wait()
        for j in range(mchunks):
            pltpu.async_copy(wd_hbm.at[idx_md.at[pl.ds(j * CH, CH)]],
                             outb.at[pl.ds(j * CH, CH)], sems[j % NBUF],
                             add=True)
        for j in range(mchunks):
            pltpu.make_async_copy(wd_hbm.at[idx_md.at[pl.ds(j * CH, CH)]],
                                  outb.at[pl.ds(j * CH, CH)],
                                  sems[j % NBUF]).wait()

        @pl.when(jnp.logical_not(is_fixer))
        def _():
            pltpu.sync_copy(outb, out_hbm.at[pl.ds(mb * b_per_w, b_per_w)])

        @pl.when(cid == 0)
        def _():
            pltpu.sync_copy(text_hbm.at[pl.ds(B + sid * t_per_w, t_per_w)],
                            idx_tt)
            pltpu.sync_copy(deps_hbm.at[pl.ds(B + sid * t_per_w, t_per_w)],
                            idx_td)

            zero = jnp.zeros((D,), jnp.float32)

            def init_acc(i, carry):
                for a in accs:
                    a[i] = zero
                return carry

            lax.fori_loop(0, CH, init_acc, 0)

            def run_table(src_hbm, idx_ref):
                def chunk_slice(c):
                    return idx_ref.at[pl.ds(pl.multiple_of(c * CH, CH), CH)]

                for b in range(NBUF):
                    pltpu.async_copy(src_hbm.at[chunk_slice(b)], accs[b],
                                     sems[b], add=True)

                def body(g, carry):
                    for b in range(NBUF):
                        pltpu.make_async_copy(src_hbm.at[chunk_slice(b)],
                                              accs[b], sems[b]).wait()
                        pltpu.async_copy(src_hbm.at[chunk_slice(g * NBUF + b)],
                                         accs[b], sems[b], add=True)
                    return carry

                lax.fori_loop(1, ngroups, body, 0)
                for b in range(NBUF):
                    pltpu.make_async_copy(src_hbm.at[chunk_slice(b)],
                                          accs[b], sems[b]).wait()

            run_table(wt_hbm, idx_tt)
            run_table(wd_hbm, idx_td)

            def red(i, carry):
                return carry + ((acc0[i] + acc1[i]) + (acc2[i] + acc3[i]))

            total = lax.fori_loop(0, CH, red, jnp.zeros((D,), jnp.float32))
            stage[0] = total
            pltpu.sync_copy(stage, spm.at[sid])

        plsc.subcore_barrier()

        @pl.when(is_fixer)
        def _():
            pltpu.sync_copy(spm, partv)

            def redp(i, carry):
                return carry + partv[i, 0, :]

            tail_total = lax.fori_loop(0, NS, redp,
                                       jnp.zeros((D,), jnp.float32))
            outb[b_per_w - 1] = outb[b_per_w - 1] + tail_total
            pltpu.sync_copy(outb, out_hbm.at[pl.ds(mb * b_per_w, b_per_w)])

    return sc_kernel


def _linearize(w):
    flat = lax.optimization_barrier(w.reshape(-1))
    return flat.reshape(w.shape)


def kernel(text, text_offsets, deps, deps_offsets, W_text, W_deps, bias):
    B = text_offsets.shape[0]
    T = text.shape[0]
    D = W_text.shape[1]
    sc_kernel = _build_sc_kernel(B, T, D)
    return sc_kernel(text.astype(jnp.int32), deps.astype(jnp.int32),
                     _linearize(W_text.astype(jnp.float32)),
                     _linearize(W_deps.astype(jnp.float32)),
                     bias.astype(jnp.float32))
